# initial kernel scaffold (unmeasured)
import jax
import jax.numpy as jnp
from jax import lax
from jax.experimental import pallas as pl
from jax.experimental.pallas import tpu as pltpu


def kernel(
    x,
):
    def body(*refs):
        pass

    out_shape = jax.ShapeDtypeStruct(..., jnp.float32)
    return pl.pallas_call(body, out_shape=out_shape)(...)


# baseline (device time: 316958 ns/iter reference)
import jax
import jax.numpy as jnp
from jax import lax
from jax.experimental import pallas as pl
from jax.experimental.pallas import tpu as pltpu

N_DEV = 4
M = 4096
N_COL = 1024


def kernel(x):
    x2 = x.reshape(M, N_DEV * N_COL)

    def body(x_hbm, out_ref, stage32, sendbuf, comm, copy_sem,
             send_sems, recv_sems):
        my_x = lax.axis_index("x")
        my_y = lax.axis_index("y")
        my_z = lax.axis_index("z")
        left = (my_z - 1) % N_DEV
        right = (my_z + 1) % N_DEV

        barrier = pltpu.get_barrier_semaphore()
        for nbr in (left, right):
            pl.semaphore_signal(
                barrier, inc=1,
                device_id=(my_x, my_y, nbr),
                device_id_type=pl.DeviceIdType.MESH,
            )
        pl.semaphore_wait(barrier, 2)

        def load_chunk(c):
            cp = pltpu.make_async_copy(
                x_hbm.at[:, pl.ds(c * N_COL, N_COL)], stage32, copy_sem)
            cp.start()
            cp.wait()

        for s in range(N_DEV - 1):
            c_send = (my_z - s - 1) % N_DEV
            load_chunk(c_send)
            if s == 0:
                sendbuf[...] = stage32[...].astype(jnp.bfloat16)
            else:
                sendbuf[...] = stage32[...].astype(jnp.bfloat16) + comm[s - 1]
            rdma = pltpu.make_async_remote_copy(
                src_ref=sendbuf,
                dst_ref=comm.at[s],
                send_sem=send_sems.at[s],
                recv_sem=recv_sems.at[s],
                device_id=(my_x, my_y, right),
                device_id_type=pl.DeviceIdType.MESH,
            )
            rdma.start()
            rdma.wait()

        load_chunk(my_z)
        out_ref[...] = stage32[...].astype(jnp.bfloat16) + comm[N_DEV - 2]

    return pl.pallas_call(
        body,
        out_shape=jax.ShapeDtypeStruct((M, N_COL), jnp.bfloat16),
        in_specs=[pl.BlockSpec(memory_space=pltpu.MemorySpace.HBM)],
        out_specs=pl.BlockSpec(memory_space=pltpu.MemorySpace.VMEM),
        scratch_shapes=[
            pltpu.MemorySpace.VMEM((M, N_COL), jnp.float32),
            pltpu.MemorySpace.VMEM((M, N_COL), jnp.bfloat16),
            pltpu.MemorySpace.VMEM((N_DEV - 1, M, N_COL), jnp.bfloat16),
            pltpu.SemaphoreType.DMA,
            pltpu.SemaphoreType.DMA((N_DEV - 1,)),
            pltpu.SemaphoreType.DMA((N_DEV - 1,)),
        ],
        compiler_params=pltpu.CompilerParams(
            collective_id=0,
            vmem_limit_bytes=60 * 1024 * 1024,
        ),
    )(x2)


# device time: 292473 ns/iter; 1.0837x vs baseline; 1.0837x over previous
import jax
import jax.numpy as jnp
from jax import lax
from jax.experimental import pallas as pl
from jax.experimental.pallas import tpu as pltpu

N_DEV = 4
M = 4096
N_COL = 1024
SB = 2
MB = M // SB


def kernel(x):
    x2 = x.reshape(M, N_DEV * N_COL)

    def body(x_hbm, out_ref, stage32, sendbuf, comm, copy_sem,
             send_sems, recv_sems):
        my_x = lax.axis_index("x")
        my_y = lax.axis_index("y")
        my_z = lax.axis_index("z")
        left = (my_z - 1) % N_DEV
        right = (my_z + 1) % N_DEV

        barrier = pltpu.get_barrier_semaphore()
        for nbr in (left, right):
            pl.semaphore_signal(
                barrier, inc=1,
                device_id=(my_x, my_y, nbr),
                device_id_type=pl.DeviceIdType.MESH,
            )
        pl.semaphore_wait(barrier, 2)

        def chunk_load(c):
            cp = pltpu.make_async_copy(
                x_hbm.at[:, pl.ds(c * N_COL, N_COL)], stage32, copy_sem)
            cp.start()
            return cp

        def hop_rdma(s, b):
            return pltpu.make_async_remote_copy(
                src_ref=sendbuf.at[b],
                dst_ref=comm.at[s, b],
                send_sem=send_sems.at[s, b],
                recv_sem=recv_sems.at[s, b],
                device_id=(my_x, my_y, right),
                device_id_type=pl.DeviceIdType.MESH,
            )

        def rows(b):
            return pl.ds(b * MB, MB)

        chunk_load((my_z - 1) % N_DEV).wait()
        rdmas = {}
        for b in range(SB):
            sendbuf[b] = stage32[rows(b)].astype(jnp.bfloat16)
            rdmas[0, b] = hop_rdma(0, b)
            rdmas[0, b].start()
        cp = chunk_load((my_z - 2) % N_DEV)

        for s in range(1, N_DEV - 1):
            cp.wait()
            for b in range(SB):
                rdmas[s - 1, b].wait_recv()
                rdmas[s - 1, b].wait_send()
                sendbuf[b] = (stage32[rows(b)].astype(jnp.bfloat16)
                              + comm[s - 1, b])
                rdmas[s, b] = hop_rdma(s, b)
                rdmas[s, b].start()
            cp = chunk_load((my_z - s - 2) % N_DEV)

        cp.wait()
        for b in range(SB):
            rdmas[N_DEV - 2, b].wait_recv()
            out_ref[rows(b)] = (stage32[rows(b)].astype(jnp.bfloat16)
                                + comm[N_DEV - 2, b])
            rdmas[N_DEV - 2, b].wait_send()

    return pl.pallas_call(
        body,
        out_shape=jax.ShapeDtypeStruct((M, N_COL), jnp.bfloat16),
        in_specs=[pl.BlockSpec(memory_space=pltpu.MemorySpace.HBM)],
        out_specs=pl.BlockSpec(memory_space=pltpu.MemorySpace.VMEM),
        scratch_shapes=[
            pltpu.MemorySpace.VMEM((M, N_COL), jnp.float32),
            pltpu.MemorySpace.VMEM((SB, MB, N_COL), jnp.bfloat16),
            pltpu.MemorySpace.VMEM(
                (N_DEV - 1, SB, MB, N_COL), jnp.bfloat16),
            pltpu.SemaphoreType.DMA,
            pltpu.SemaphoreType.DMA((N_DEV - 1, SB)),
            pltpu.SemaphoreType.DMA((N_DEV - 1, SB)),
        ],
        compiler_params=pltpu.CompilerParams(
            collective_id=0,
            vmem_limit_bytes=60 * 1024 * 1024,
        ),
    )(x2)


# device time: 292465 ns/iter; 1.0837x vs baseline; 1.0000x over previous
import jax
import jax.numpy as jnp
from jax import lax
from jax.experimental import pallas as pl
from jax.experimental.pallas import tpu as pltpu

N_DEV = 4
M = 4096
N_COL = 1024
SB = 4
MB = M // SB


def kernel(x):
    x2 = x.reshape(M, N_DEV * N_COL)

    def body(x_hbm, out_ref, stage32, sendbuf, comm, copy_sem,
             send_sems, recv_sems):
        my_x = lax.axis_index("x")
        my_y = lax.axis_index("y")
        my_z = lax.axis_index("z")
        left = (my_z - 1) % N_DEV
        right = (my_z + 1) % N_DEV

        def chunk_load(c):
            cp = pltpu.make_async_copy(
                x_hbm.at[:, pl.ds(c * N_COL, N_COL)], stage32, copy_sem)
            cp.start()
            return cp

        def hop_rdma(s, b):
            return pltpu.make_async_remote_copy(
                src_ref=sendbuf.at[b],
                dst_ref=comm.at[s, b],
                send_sem=send_sems.at[s, b],
                recv_sem=recv_sems.at[s, b],
                device_id=(my_x, my_y, right),
                device_id_type=pl.DeviceIdType.MESH,
            )

        def rows(b):
            return pl.ds(b * MB, MB)

        barrier = pltpu.get_barrier_semaphore()
        for nbr in (left, right):
            pl.semaphore_signal(
                barrier, inc=1,
                device_id=(my_x, my_y, nbr),
                device_id_type=pl.DeviceIdType.MESH,
            )

        chunk_load((my_z - 1) % N_DEV).wait()
        rdmas = {}
        for b in range(SB):
            sendbuf[b] = stage32[rows(b)].astype(jnp.bfloat16)
        pl.semaphore_wait(barrier, 2)
        for b in range(SB):
            rdmas[0, b] = hop_rdma(0, b)
            rdmas[0, b].start()
        cp = chunk_load((my_z - 2) % N_DEV)

        for s in range(1, N_DEV - 1):
            cp.wait()
            for b in range(SB):
                rdmas[s - 1, b].wait_recv()
                rdmas[s - 1, b].wait_send()
                sendbuf[b] = (stage32[rows(b)].astype(jnp.bfloat16)
                              + comm[s - 1, b])
                rdmas[s, b] = hop_rdma(s, b)
                rdmas[s, b].start()
            cp = chunk_load((my_z - s - 2) % N_DEV)

        cp.wait()
        for b in range(SB):
            rdmas[N_DEV - 2, b].wait_recv()
            out_ref[rows(b)] = (stage32[rows(b)].astype(jnp.bfloat16)
                                + comm[N_DEV - 2, b])
            rdmas[N_DEV - 2, b].wait_send()

    return pl.pallas_call(
        body,
        out_shape=jax.ShapeDtypeStruct((M, N_COL), jnp.bfloat16),
        in_specs=[pl.BlockSpec(memory_space=pltpu.MemorySpace.HBM)],
        out_specs=pl.BlockSpec(memory_space=pltpu.MemorySpace.VMEM),
        scratch_shapes=[
            pltpu.MemorySpace.VMEM((M, N_COL), jnp.float32),
            pltpu.MemorySpace.VMEM((SB, MB, N_COL), jnp.bfloat16),
            pltpu.MemorySpace.VMEM(
                (N_DEV - 1, SB, MB, N_COL), jnp.bfloat16),
            pltpu.SemaphoreType.DMA,
            pltpu.SemaphoreType.DMA((N_DEV - 1, SB)),
            pltpu.SemaphoreType.DMA((N_DEV - 1, SB)),
        ],
        compiler_params=pltpu.CompilerParams(
            collective_id=0,
            vmem_limit_bytes=60 * 1024 * 1024,
        ),
    )(x2)
